# baseline (speedup 1.0000x reference)
import functools

import jax
import jax.numpy as jnp
from jax import lax
from jax.experimental import pallas as pl
from jax.experimental.pallas import tpu as pltpu
from jax.experimental.pallas import tpu_sc as plsc

Q = 1024
D = 128
K_REAL = 100000
KT = 1024
K_PAD = 100352
N_TILES = K_PAD // KT
CW = 128
NCH = K_PAD // CW
NCH_PAD = 1024
NSEL = 128
NEG = float("-inf")


def _matmul_body(q_ref, k_ref, s_ref, m_ref):
    s = lax.dot_general(
        q_ref[...], k_ref[...],
        dimension_numbers=(((1,), (1,)), ((), ())),
        preferred_element_type=jnp.float32,
    )
    col = pl.program_id(0) * KT + lax.broadcasted_iota(jnp.int32, s.shape, 1)
    s = jnp.where(col < K_REAL, s, NEG)
    s_ref[...] = s
    m_ref[...] = jnp.max(s.reshape(Q, KT // CW, CW), axis=2)[None]


def _scores_and_chunkmax(q, keys_p):
    return pl.pallas_call(
        _matmul_body,
        grid=(N_TILES,),
        in_specs=[
            pl.BlockSpec((Q, D), lambda i: (0, 0)),
            pl.BlockSpec((KT, D), lambda i: (i, 0)),
        ],
        out_specs=[
            pl.BlockSpec((Q, KT), lambda i: (0, i)),
            pl.BlockSpec((1, Q, KT // CW), lambda i: (i, 0, 0)),
        ],
        out_shape=[
            jax.ShapeDtypeStruct((Q, K_PAD), jnp.float32),
            jax.ShapeDtypeStruct((N_TILES, Q, KT // CW), jnp.float32),
        ],
    )(q, keys_p)


def _ce_s(v, ix, d, desc):
    ax = v.ndim - 1
    lane = lax.broadcasted_iota(jnp.int32, v.shape, ax)
    low = (lane & d) == 0
    pv = jnp.where(low, pltpu.roll(v, 128 - d, ax), pltpu.roll(v, d, ax))
    pi = jnp.where(low, pltpu.roll(ix, 128 - d, ax), pltpu.roll(ix, d, ax))
    win = (v > pv) | ((v == pv) & (ix < pi))
    keep = win ^ low ^ desc
    return jnp.where(keep, v, pv), jnp.where(keep, ix, pi)


def _sort_body(v_ref, ix_ref, ov_ref, oi_ref):
    v = v_ref[...]
    ix = ix_ref[...]
    run = lax.broadcasted_iota(jnp.int32, v.shape, 1)
    lane = lax.broadcasted_iota(jnp.int32, v.shape, 2)
    g = run * 128 + lane
    for k in range(1, 8):
        desc = ((g >> k) & 1) == 0
        for j in range(k, 0, -1):
            v, ix = _ce_s(v, ix, 1 << (j - 1), desc)
    ov_ref[...] = v
    oi_ref[...] = ix


def _sort_runs(vals, ixs):
    r = vals.shape[1]
    rg = min(r, 8)
    return pl.pallas_call(
        _sort_body,
        grid=(Q // 8, r // rg),
        in_specs=[
            pl.BlockSpec((8, rg, 128), lambda i, b: (i, b, 0)),
            pl.BlockSpec((8, rg, 128), lambda i, b: (i, b, 0)),
        ],
        out_specs=[
            pl.BlockSpec((8, rg, 128), lambda i, b: (i, b, 0)),
            pl.BlockSpec((8, rg, 128), lambda i, b: (i, b, 0)),
        ],
        out_shape=[
            jax.ShapeDtypeStruct((Q, r, 128), jnp.float32),
            jax.ShapeDtypeStruct((Q, r, 128), jnp.int32),
        ],
    )(vals, ixs)


def _merge_body(v_ref, ix_ref, ov_ref, oi_ref):
    av, bv = v_ref[:, 0, 0, :], v_ref[:, 0, 1, :]
    ai, bi = ix_ref[:, 0, 0, :], ix_ref[:, 0, 1, :]
    win = (av > bv) | ((av == bv) & (ai < bi))
    v = jnp.where(win, av, bv)
    ix = jnp.where(win, ai, bi)
    lane = lax.broadcasted_iota(jnp.int32, v.shape, 1)
    desc = ((lane & 0) + (pl.program_id(1) & 1)) == 0
    for j in range(7, 0, -1):
        v, ix = _ce_s(v, ix, 1 << (j - 1), desc)
    ov_ref[:, 0, 0, :] = v
    oi_ref[:, 0, 0, :] = ix


def _merge_level(vals, ixs):
    r = vals.shape[1]
    v4 = vals.reshape(Q, r // 2, 2, 128)
    i4 = ixs.reshape(Q, r // 2, 2, 128)
    ov, oi = pl.pallas_call(
        _merge_body,
        grid=(Q // 32, r // 2),
        in_specs=[
            pl.BlockSpec((32, 1, 2, 128), lambda i, p: (i, p, 0, 0)),
            pl.BlockSpec((32, 1, 2, 128), lambda i, p: (i, p, 0, 0)),
        ],
        out_specs=[
            pl.BlockSpec((32, 1, 1, 128), lambda i, p: (i, p, 0, 0)),
            pl.BlockSpec((32, 1, 1, 128), lambda i, p: (i, p, 0, 0)),
        ],
        out_shape=[
            jax.ShapeDtypeStruct((Q, r // 2, 1, 128), jnp.float32),
            jax.ShapeDtypeStruct((Q, r // 2, 1, 128), jnp.int32),
        ],
    )(v4, i4)
    return ov.reshape(Q, r // 2, 128), oi.reshape(Q, r // 2, 128)


def _topk128(vals, ixs):
    v, ix = _sort_runs(vals, ixs)
    while v.shape[1] > 1:
        v, ix = _merge_level(v, ix)
    return v[:, 0, :], ix[:, 0, :]


def _sc_gather(table, idx):
    b = idx.shape[0]
    info = plsc.get_sparse_core_info()
    nw = info.num_cores * info.num_subcores
    b_per_w = b // nw
    ch = 128
    n_ch = b_per_w // ch
    mesh = plsc.VectorSubcoreMesh(core_axis_name="c", subcore_axis_name="s")

    @functools.partial(
        pl.kernel,
        mesh=mesh,
        out_type=jax.ShapeDtypeStruct((b, 128), jnp.float32),
        scratch_types=[
            pltpu.VMEM((b_per_w,), jnp.int32),
            pltpu.VMEM((ch, 128), jnp.float32),
            pltpu.SemaphoreType.DMA,
        ],
    )
    def k(table_hbm, idx_hbm, out_hbm, idx_v, rows_v, sem):
        wid = lax.axis_index("s") * info.num_cores + lax.axis_index("c")
        base = wid * b_per_w
        pltpu.sync_copy(idx_hbm.at[pl.ds(base, b_per_w)], idx_v)

        def body(c, carry):
            start = pl.multiple_of(c * ch, ch)
            pltpu.async_copy(
                table_hbm.at[idx_v.at[pl.ds(start, ch)]], rows_v, sem
            ).wait()
            pltpu.sync_copy(rows_v, out_hbm.at[pl.ds(base + start, ch)])
            return carry

        lax.fori_loop(0, n_ch, body, 0)

    return k(table, idx)


def kernel(queries, keys, topk):
    del topk
    keys_p = jnp.pad(keys, ((0, K_PAD - K_REAL), (0, 0)))
    scores, cmax3 = _scores_and_chunkmax(queries, keys_p)
    cmax = jnp.transpose(cmax3, (1, 0, 2)).reshape(Q, NCH)

    cmax_p = jnp.pad(cmax, ((0, 0), (0, NCH_PAD - NCH)), constant_values=NEG)
    cix = jnp.broadcast_to(
        jnp.arange(NCH_PAD, dtype=jnp.int32).reshape(1, NCH_PAD // 128, 128),
        (Q, NCH_PAD // 128, 128),
    )
    _, chunk_ids = _topk128(cmax_p.reshape(Q, NCH_PAD // 128, 128), cix)

    flat = (jnp.arange(Q, dtype=jnp.int32)[:, None] * NCH + chunk_ids).reshape(-1)
    cand = _sc_gather(scores.reshape(Q * NCH, CW), flat)
    cand_v = cand.reshape(Q, NSEL, CW)
    cand_ix = chunk_ids[:, :, None] * CW + jnp.arange(CW, dtype=jnp.int32)[None, None, :]

    top_vals, top_idx = _topk128(cand_v, cand_ix)
    logits = top_vals[:, :100]
    tidx = top_idx[:, :100]

    emb = _sc_gather(keys, tidx.reshape(-1))
    return logits, tidx, emb.reshape(Q, 100, D)


# speedup vs baseline: 1.2942x; 1.2942x over previous
import functools

import jax
import jax.numpy as jnp
from jax import lax
from jax.experimental import pallas as pl
from jax.experimental.pallas import tpu as pltpu
from jax.experimental.pallas import tpu_sc as plsc

Q = 1024
D = 128
K_REAL = 100000
KT = 1024
K_PAD = 100352
N_TILES = K_PAD // KT
CW = 128
NCH = K_PAD // CW
NCH_PAD = 1024
NSEL = 128
NEG = float("-inf")


def _matmul_body(q_ref, k_ref, s_ref, m_ref):
    s = lax.dot_general(
        q_ref[...], k_ref[...],
        dimension_numbers=(((1,), (1,)), ((), ())),
        preferred_element_type=jnp.float32,
    )
    col = pl.program_id(0) * KT + lax.broadcasted_iota(jnp.int32, s.shape, 1)
    s = jnp.where(col < K_REAL, s, NEG)
    s_ref[...] = s
    m_ref[...] = jnp.max(s.reshape(Q, KT // CW, CW), axis=2)[None]


def _scores_and_chunkmax(q, keys_p):
    return pl.pallas_call(
        _matmul_body,
        grid=(N_TILES,),
        in_specs=[
            pl.BlockSpec((Q, D), lambda i: (0, 0)),
            pl.BlockSpec((KT, D), lambda i: (i, 0)),
        ],
        out_specs=[
            pl.BlockSpec((Q, KT), lambda i: (0, i)),
            pl.BlockSpec((1, Q, KT // CW), lambda i: (i, 0, 0)),
        ],
        out_shape=[
            jax.ShapeDtypeStruct((Q, K_PAD), jnp.float32),
            jax.ShapeDtypeStruct((N_TILES, Q, KT // CW), jnp.float32),
        ],
    )(q, keys_p)


def _ce3_dyn(v, ix, d, desc):
    ax = len(v.shape) - 1
    lane = lax.broadcasted_iota(jnp.int32, v.shape, ax)
    low = (lane & d) == 0
    pv = jnp.where(low, pltpu.roll(v, 128 - d, ax), pltpu.roll(v, d, ax))
    pi = jnp.where(low, pltpu.roll(ix, 128 - d, ax), pltpu.roll(ix, d, ax))
    win = (v > pv) | ((v == pv) & (ix < pi))
    keep = win ^ low ^ desc
    return jnp.where(keep, v, pv), jnp.where(keep, ix, pi)


def _sort_body(v_ref, ix_ref, ov_ref, oi_ref):
    qt, m = v_ref.shape
    r = m // 128
    v = v_ref[...].reshape(qt, r, 128)
    ix = ix_ref[...].reshape(qt, r, 128)
    run = lax.broadcasted_iota(jnp.int32, v.shape, 1)
    lane = lax.broadcasted_iota(jnp.int32, v.shape, 2)
    g = run * 128 + lane
    for k in range(1, 8):
        desc = ((g >> k) & 1) == 0

        def stage(t, carry, k=k, desc=desc):
            d = jnp.int32(1) << (k - 1 - t)
            return _ce3_dyn(*carry, d, desc)

        v, ix = lax.fori_loop(0, k, stage, (v, ix))
    ov_ref[...] = v
    oi_ref[...] = ix


def _sort_runs(vals, ixs, q_tile):
    m = vals.shape[1]
    r = m // 128
    return pl.pallas_call(
        _sort_body,
        grid=(Q // q_tile,),
        in_specs=[
            pl.BlockSpec((q_tile, m), lambda i: (i, 0)),
            pl.BlockSpec((q_tile, m), lambda i: (i, 0)),
        ],
        out_specs=[
            pl.BlockSpec((q_tile, r, 128), lambda i: (i, 0, 0)),
            pl.BlockSpec((q_tile, r, 128), lambda i: (i, 0, 0)),
        ],
        out_shape=[
            jax.ShapeDtypeStruct((Q, r, 128), jnp.float32),
            jax.ShapeDtypeStruct((Q, r, 128), jnp.int32),
        ],
    )(vals, ixs)


def _merge_body(v_ref, ix_ref, ov_ref, oi_ref):
    av, bv = v_ref[:, 0, 0, :], v_ref[:, 0, 1, :]
    ai, bi = ix_ref[:, 0, 0, :], ix_ref[:, 0, 1, :]
    win = (av > bv) | ((av == bv) & (ai < bi))
    v = jnp.where(win, av, bv)
    ix = jnp.where(win, ai, bi)
    lane = lax.broadcasted_iota(jnp.int32, v.shape, 1)
    desc = ((lane & 0) + (pl.program_id(1) & 1)) == 0
    for j in range(7, 0, -1):
        v, ix = _ce3_dyn(v, ix, jnp.int32(1 << (j - 1)), desc)
    ov_ref[:, 0, 0, :] = v
    oi_ref[:, 0, 0, :] = ix


def _merge_level(vals, ixs):
    r = vals.shape[1]
    v4 = vals.reshape(Q, r // 2, 2, 128)
    i4 = ixs.reshape(Q, r // 2, 2, 128)
    ov, oi = pl.pallas_call(
        _merge_body,
        grid=(Q // 32, r // 2),
        in_specs=[
            pl.BlockSpec((32, 1, 2, 128), lambda i, p: (i, p, 0, 0)),
            pl.BlockSpec((32, 1, 2, 128), lambda i, p: (i, p, 0, 0)),
        ],
        out_specs=[
            pl.BlockSpec((32, 1, 1, 128), lambda i, p: (i, p, 0, 0)),
            pl.BlockSpec((32, 1, 1, 128), lambda i, p: (i, p, 0, 0)),
        ],
        out_shape=[
            jax.ShapeDtypeStruct((Q, r // 2, 1, 128), jnp.float32),
            jax.ShapeDtypeStruct((Q, r // 2, 1, 128), jnp.int32),
        ],
    )(v4, i4)
    return ov.reshape(Q, r // 2, 128), oi.reshape(Q, r // 2, 128)


def _topk128(vals, ixs, q_tile):
    v, ix = _sort_runs(vals, ixs, q_tile)
    while v.shape[1] > 1:
        v, ix = _merge_level(v, ix)
    return v[:, 0, :], ix[:, 0, :]


def _sc_gather(table, idx):
    b = idx.shape[0]
    info = plsc.get_sparse_core_info()
    nw = info.num_cores * info.num_subcores
    b_per_w = b // nw
    ch = 128
    n_ch = b_per_w // ch
    mesh = plsc.VectorSubcoreMesh(core_axis_name="c", subcore_axis_name="s")

    @functools.partial(
        pl.kernel,
        mesh=mesh,
        out_type=jax.ShapeDtypeStruct((b, 128), jnp.float32),
        scratch_types=[
            pltpu.VMEM((b_per_w,), jnp.int32),
            pltpu.VMEM((ch, 128), jnp.float32),
            pltpu.SemaphoreType.DMA,
        ],
    )
    def k(table_hbm, idx_hbm, out_hbm, idx_v, rows_v, sem):
        wid = lax.axis_index("s") * info.num_cores + lax.axis_index("c")
        base = wid * b_per_w
        pltpu.sync_copy(idx_hbm.at[pl.ds(base, b_per_w)], idx_v)

        def body(c, carry):
            start = pl.multiple_of(c * ch, ch)
            pltpu.async_copy(
                table_hbm.at[idx_v.at[pl.ds(start, ch)]], rows_v, sem
            ).wait()
            pltpu.sync_copy(rows_v, out_hbm.at[pl.ds(base + start, ch)])
            return carry

        lax.fori_loop(0, n_ch, body, 0)

    return k(table, idx)


def kernel(queries, keys, topk):
    del topk
    keys_p = jnp.pad(keys, ((0, K_PAD - K_REAL), (0, 0)))
    scores, cmax3 = _scores_and_chunkmax(queries, keys_p)
    cmax = jnp.transpose(cmax3, (1, 0, 2)).reshape(Q, NCH)

    cmax_p = jnp.pad(cmax, ((0, 0), (0, NCH_PAD - NCH)), constant_values=NEG)
    lane = jnp.arange(NCH_PAD, dtype=jnp.int32)
    _, chunk_ids = _topk128(cmax_p, jnp.broadcast_to(lane, (Q, NCH_PAD)), 64)

    flat = (jnp.arange(Q, dtype=jnp.int32)[:, None] * NCH + chunk_ids).reshape(-1)
    cand = _sc_gather(scores.reshape(Q * NCH, CW), flat)
    cand_v = cand.reshape(Q, NSEL * CW)
    cand_ix = (
        chunk_ids[:, :, None] * CW + jnp.arange(CW, dtype=jnp.int32)[None, None, :]
    ).reshape(Q, NSEL * CW)

    top_vals, top_idx = _topk128(cand_v, cand_ix, 16)
    logits = top_vals[:, :100]
    tidx = top_idx[:, :100]

    emb = _sc_gather(keys, tidx.reshape(-1))
    return logits, tidx, emb.reshape(Q, 100, D)
